# baseline (speedup 1.0000x reference)
import jax
import jax.numpy as jnp
from jax import lax
from jax.experimental import pallas as pl
from jax.experimental.pallas import tpu as pltpu
from jax.experimental.pallas import tpu_sc as plsc

SEQ_LEN = 2048
N_EMBD = 768
NUM_CORES = 2
NUM_SUBCORES = 16
NUM_WORKERS = NUM_CORES * NUM_SUBCORES
ROWS_PER_WORKER = SEQ_LEN // NUM_WORKERS
LANES = 16
VECS_PER_ROW = N_EMBD // LANES
CHUNK = 16
NCHUNKS = ROWS_PER_WORKER // CHUNK
NBUF = 2


def _emb_body(ids_hbm, wte_hbm, wpe_hbm, out_hbm, idx_v, rows_v, wpe_v,
              sum_v, gat_sems, lin_sems, out_sems):
    wid = lax.axis_index("s") * NUM_CORES + lax.axis_index("c")
    base = wid * ROWS_PER_WORKER

    pltpu.sync_copy(ids_hbm.at[pl.ds(base, ROWS_PER_WORKER)], idx_v)

    def gat_copy(k):
        b = k % NBUF
        return pltpu.make_async_copy(
            wte_hbm.at[idx_v.at[pl.ds(k * CHUNK, CHUNK)]], rows_v.at[b],
            gat_sems[b])

    def lin_copy(k):
        b = k % NBUF
        return pltpu.make_async_copy(
            wpe_hbm.at[pl.ds(base + k * CHUNK, CHUNK)], wpe_v.at[b],
            lin_sems[b])

    def out_copy(k):
        b = k % NBUF
        return pltpu.make_async_copy(
            sum_v.at[b], out_hbm.at[pl.ds(base + k * CHUNK, CHUNK)],
            out_sems[b])

    def start_chunk(k):
        gat_copy(k).start()
        lin_copy(k).start()

    for k in range(NBUF):
        start_chunk(k)

    for k in range(NCHUNKS):
        b = k % NBUF
        gat_copy(k).wait()
        lin_copy(k).wait()
        if k >= NBUF:
            out_copy(k - NBUF).wait()

        def add_row(j, carry, b=b):
            for i in range(VECS_PER_ROW):
                sl = pl.ds(i * LANES, LANES)
                sum_v[b, j, sl] = rows_v[b, j, sl] + wpe_v[b, j, sl]
            return carry

        lax.fori_loop(0, CHUNK, add_row, 0, unroll=False)

        out_copy(k).start()
        if k + NBUF < NCHUNKS:
            start_chunk(k + NBUF)

    for k in range(NCHUNKS - NBUF, NCHUNKS):
        out_copy(k).wait()


@jax.jit
def _embedding(input_ids, wte, wpe):
    mesh = plsc.VectorSubcoreMesh(core_axis_name="c", subcore_axis_name="s")
    run = pl.kernel(
        _emb_body,
        out_type=jax.ShapeDtypeStruct((SEQ_LEN, N_EMBD), jnp.float32),
        mesh=mesh,
        scratch_types=[
            pltpu.VMEM((ROWS_PER_WORKER,), jnp.int32),
            pltpu.VMEM((NBUF, CHUNK, N_EMBD), jnp.float32),
            pltpu.VMEM((NBUF, CHUNK, N_EMBD), jnp.float32),
            pltpu.VMEM((NBUF, CHUNK, N_EMBD), jnp.float32),
            [pltpu.SemaphoreType.DMA] * NBUF,
            [pltpu.SemaphoreType.DMA] * NBUF,
            [pltpu.SemaphoreType.DMA] * NBUF,
        ],
    )
    return run(input_ids, wte, wpe)


def kernel(input_ids, wte, wpe):
    out = _embedding(input_ids.astype(jnp.int32), wte, wpe)
    return out[None, :, :]
